# baseline (device time: 300561 ns/iter reference)
import jax
import jax.numpy as jnp
from jax import lax
from jax.experimental import pallas as pl
from jax.experimental.pallas import tpu as pltpu

N_DEV = 4
M, N = 16384, 1024
HALF = M // 2
QUART = HALF // 2
EIGHTH = HALF // 4
CHUNK = 512
NPIECE = QUART // CHUNK

PIECE2 = 1024

RS2_0, RS2_1, AG1_0, AG1_1, AG2A, AG2B_0, AG2B_1 = range(7)


def kernel(x):
    def body(x_hbm, out_hbm, acc, rs1_buf, rs2_buf, stage,
             stage_sems, copy_sems, rs1_send, rs1_recv, m_send, m_recv):
        r = lax.axis_index("i")
        nbr_a = r ^ 1
        nbr_b = 3 - r

        barrier = pltpu.get_barrier_semaphore()
        for nbr in (nbr_a, nbr_b):
            pl.semaphore_signal(
                barrier, inc=1,
                device_id=(nbr,), device_id_type=pl.DeviceIdType.MESH,
            )
        pl.semaphore_wait(barrier, 2)

        halves = []
        for h in range(2):
            base = h * HALF
            if h == 0:
                k1 = (r ^ (r >> 1)) & 1
                k2 = (r >> 1) & 1
                partners = (nbr_a, nbr_b, nbr_b, nbr_a)
            else:
                k1 = (r >> 1) & 1
                k2 = r & 1
                partners = (nbr_b, nbr_a, nbr_a, nbr_b)
            off1_keep = base + k1 * QUART
            off1_send = base + (1 - k1) * QUART
            off2_keep = off1_keep + k2 * EIGHTH
            off2_send = off1_keep + (1 - k2) * EIGHTH
            first_rel = (1 - k2) * EIGHTH if h == 0 else k2 * EIGHTH
            second_rel = k2 * EIGHTH if h == 0 else (1 - k2) * EIGHTH
            halves.append(dict(
                partners=partners, k2=k2,
                off1_keep=off1_keep, off1_send=off1_send,
                off2_keep=off2_keep, off2_send=off2_send,
                first_rel=first_rel, second_rel=second_rel,
            ))

        def remote(src_slice, dst_slice, s_sem, r_sem, partner):
            rdma = pltpu.make_async_remote_copy(
                src_ref=src_slice, dst_ref=dst_slice,
                send_sem=s_sem, recv_sem=r_sem,
                device_id=(partner,),
                device_id_type=pl.DeviceIdType.MESH,
            )
            rdma.start()
            return rdma

        NQ = NPIECE // 2
        entries = []
        for p in range(NPIECE):
            for h, cfg in enumerate(halves):
                rel = (cfg["first_rel"] + p * CHUNK if p < NPIECE // 2
                       else cfg["second_rel"] + (p - NPIECE // 2) * CHUNK)
                send = (h, p // 2, rel - CHUNK) if p % 2 == 1 else None
                entries.append((cfg["off1_send"] + rel, send))
        for cfg in halves:
            for c in range(NPIECE):
                entries.append((cfg["off1_keep"] + c * CHUNK, None))

        rs1_rdmas = [[None] * NQ, [None] * NQ]
        cps = {}

        def start_chunk(i):
            cp = pltpu.make_async_copy(
                x_hbm.at[pl.ds(entries[i][0], CHUNK)],
                stage.at[i % 2],
                stage_sems.at[i % 2],
            )
            cp.start()
            cps[i] = cp

        start_chunk(0)
        for i, (row, send) in enumerate(entries):
            if i + 1 < len(entries):
                start_chunk(i + 1)
            cps[i].wait()
            acc[pl.ds(row, CHUNK), :] = stage[i % 2].astype(jnp.bfloat16)
            if send is not None:
                h, q, rel = send
                rs1_rdmas[h][q] = remote(
                    acc.at[pl.ds(row - CHUNK, PIECE2)],
                    rs1_buf.at[h, pl.ds(rel, PIECE2)],
                    rs1_send.at[h, q], rs1_recv.at[h, q],
                    halves[h]["partners"][0],
                )

        rs2p = [[None, None], [None, None]]
        for q in range(2):
            for h, cfg in enumerate(halves):
                rs1_rdmas[h][q].wait_recv()
                row = cfg["off2_send"] + q * PIECE2
                acc[pl.ds(row, PIECE2), :] = (
                    acc[pl.ds(row, PIECE2), :]
                    + rs1_buf[h, pl.ds((1 - cfg["k2"]) * EIGHTH + q * PIECE2,
                                       PIECE2), :]
                )
                rs2p[h][q] = remote(
                    acc.at[pl.ds(row, PIECE2)],
                    rs2_buf.at[h, pl.ds(q * PIECE2, PIECE2)],
                    m_send.at[h, RS2_0 + q], m_recv.at[h, RS2_0 + q],
                    cfg["partners"][1],
                )
        for q in range(2, NQ):
            for h, cfg in enumerate(halves):
                rs1_rdmas[h][q].wait_recv()
                row = cfg["off2_keep"] + (q - 2) * PIECE2
                acc[pl.ds(row, PIECE2), :] = (
                    acc[pl.ds(row, PIECE2), :]
                    + rs1_buf[h, pl.ds(cfg["k2"] * EIGHTH + (q - 2) * PIECE2,
                                       PIECE2), :]
                )

        ag1p = [[None, None], [None, None]]
        ag2a = [None, None]
        own_cp = [None, None]
        for j in range(2):
            for h, cfg in enumerate(halves):
                rs2p[h][j].wait_recv()
                row = cfg["off2_keep"] + j * PIECE2
                acc[pl.ds(row, PIECE2), :] = (
                    acc[pl.ds(row, PIECE2), :]
                    + rs2_buf[h, pl.ds(j * PIECE2, PIECE2), :]
                )
                ag1p[h][j] = remote(
                    acc.at[pl.ds(row, PIECE2)],
                    out_hbm.at[pl.ds(row, PIECE2)],
                    m_send.at[h, AG1_0 + j], m_recv.at[h, AG1_0 + j],
                    cfg["partners"][2],
                )
                if j == 1:
                    ag2a[h] = remote(
                        acc.at[pl.ds(cfg["off2_keep"], EIGHTH)],
                        out_hbm.at[pl.ds(cfg["off2_keep"], EIGHTH)],
                        m_send.at[h, AG2A], m_recv.at[h, AG2A],
                        cfg["partners"][3],
                    )
                    own_cp[h] = pltpu.make_async_copy(
                        acc.at[pl.ds(cfg["off2_keep"], EIGHTH)],
                        out_hbm.at[pl.ds(cfg["off2_keep"], EIGHTH)],
                        copy_sems.at[h],
                    )
                    own_cp[h].start()

        ag2bp = [[None, None], [None, None]]
        for j in range(2):
            for h, cfg in enumerate(halves):
                ag1p[h][j].wait_recv()
                row = cfg["off2_send"] + j * PIECE2
                ag2bp[h][j] = remote(
                    out_hbm.at[pl.ds(row, PIECE2)],
                    out_hbm.at[pl.ds(row, PIECE2)],
                    m_send.at[h, AG2B_0 + j], m_recv.at[h, AG2B_0 + j],
                    cfg["partners"][3],
                )

        for h in range(2):
            ag2a[h].wait_recv()
            for j in range(2):
                ag2bp[h][j].wait_recv()
            own_cp[h].wait()
        for h in range(2):
            for q in range(NQ):
                rs1_rdmas[h][q].wait_send()
            for j in range(2):
                for rdma in (rs2p[h][j], ag1p[h][j], ag2bp[h][j]):
                    rdma.wait_send()
            ag2a[h].wait_send()

    return pl.pallas_call(
        body,
        out_shape=jax.ShapeDtypeStruct((M, N), jnp.bfloat16),
        in_specs=[pl.BlockSpec(memory_space=pl.ANY)],
        out_specs=pl.BlockSpec(memory_space=pl.ANY),
        scratch_shapes=[
            pltpu.VMEM((M, N), jnp.bfloat16),
            pltpu.VMEM((2, QUART, N), jnp.bfloat16),
            pltpu.VMEM((2, EIGHTH, N), jnp.bfloat16),
            pltpu.VMEM((2, CHUNK, N), jnp.float32),
            pltpu.SemaphoreType.DMA((2,)),
            pltpu.SemaphoreType.DMA((2,)),
            pltpu.SemaphoreType.DMA((2, NPIECE)),
            pltpu.SemaphoreType.DMA((2, NPIECE)),
            pltpu.SemaphoreType.DMA((2, 7)),
            pltpu.SemaphoreType.DMA((2, 7)),
        ],
        compiler_params=pltpu.CompilerParams(
            collective_id=0,
            vmem_limit_bytes=63 * 1024 * 1024,
        ),
    )(x)


# device time: 298746 ns/iter; 1.0061x vs baseline; 1.0061x over previous
import jax
import jax.numpy as jnp
from jax import lax
from jax.experimental import pallas as pl
from jax.experimental.pallas import tpu as pltpu

N_DEV = 4
M, N = 16384, 1024
HALF = M // 2
QUART = HALF // 2
EIGHTH = HALF // 4
CHUNK = 512
NPIECE = QUART // CHUNK

PIECE2 = 1024

RS2_0, RS2_1, AG1_0, AG1_1, AG2A, AG2B_0, AG2B_1 = range(7)


def kernel(x):
    def body(x_hbm, out_hbm, acc, rs1_buf, rs2_buf, stage,
             stage_sems, copy_sems, rs1_send, rs1_recv, m_send, m_recv):
        r = lax.axis_index("i")
        nbr_a = r ^ 1
        nbr_b = 3 - r

        barrier = pltpu.get_barrier_semaphore()
        for nbr in (nbr_a, nbr_b):
            pl.semaphore_signal(
                barrier, inc=1,
                device_id=(nbr,), device_id_type=pl.DeviceIdType.MESH,
            )
        pl.semaphore_wait(barrier, 2)

        halves = []
        for h in range(2):
            base = h * HALF
            if h == 0:
                k1 = (r ^ (r >> 1)) & 1
                k2 = (r >> 1) & 1
                partners = (nbr_a, nbr_b, nbr_b, nbr_a)
            else:
                k1 = (r >> 1) & 1
                k2 = r & 1
                partners = (nbr_b, nbr_a, nbr_a, nbr_b)
            off1_keep = base + k1 * QUART
            off1_send = base + (1 - k1) * QUART
            off2_keep = off1_keep + k2 * EIGHTH
            off2_send = off1_keep + (1 - k2) * EIGHTH
            first_rel = (1 - k2) * EIGHTH if h == 0 else k2 * EIGHTH
            second_rel = k2 * EIGHTH if h == 0 else (1 - k2) * EIGHTH
            halves.append(dict(
                partners=partners, k2=k2,
                off1_keep=off1_keep, off1_send=off1_send,
                off2_keep=off2_keep, off2_send=off2_send,
                first_rel=first_rel, second_rel=second_rel,
            ))

        def remote(src_slice, dst_slice, s_sem, r_sem, partner):
            rdma = pltpu.make_async_remote_copy(
                src_ref=src_slice, dst_ref=dst_slice,
                send_sem=s_sem, recv_sem=r_sem,
                device_id=(partner,),
                device_id_type=pl.DeviceIdType.MESH,
            )
            rdma.start()
            return rdma

        entries = []
        for p in range(NPIECE):
            for h, cfg in enumerate(halves):
                rel = (cfg["first_rel"] + p * CHUNK if p < NPIECE // 2
                       else cfg["second_rel"] + (p - NPIECE // 2) * CHUNK)
                entries.append((cfg["off1_send"] + rel, (h, p, rel)))
        for cfg in halves:
            for c in range(NPIECE):
                entries.append((cfg["off1_keep"] + c * CHUNK, None))

        rs1_rdmas = [[None] * NPIECE, [None] * NPIECE]
        cps = {}

        def start_chunk(i):
            cp = pltpu.make_async_copy(
                x_hbm.at[pl.ds(entries[i][0], CHUNK)],
                stage.at[i % 2],
                stage_sems.at[i % 2],
            )
            cp.start()
            cps[i] = cp

        start_chunk(0)
        for i, (row, send) in enumerate(entries):
            if i + 1 < len(entries):
                start_chunk(i + 1)
            cps[i].wait()
            acc[pl.ds(row, CHUNK), :] = stage[i % 2].astype(jnp.bfloat16)
            if send is not None:
                h, p, rel = send
                rs1_rdmas[h][p] = remote(
                    acc.at[pl.ds(row, CHUNK)],
                    rs1_buf.at[h, pl.ds(rel, CHUNK)],
                    rs1_send.at[h, p], rs1_recv.at[h, p],
                    halves[h]["partners"][0],
                )

        rs2p = [[None, None], [None, None]]
        for p in range(NPIECE // 2):
            for h, cfg in enumerate(halves):
                rs1_rdmas[h][p].wait_recv()
                row = cfg["off2_send"] + p * CHUNK
                acc[pl.ds(row, CHUNK), :] = (
                    acc[pl.ds(row, CHUNK), :]
                    + rs1_buf[h, pl.ds((1 - cfg["k2"]) * EIGHTH + p * CHUNK,
                                       CHUNK), :]
                )
            if p % 2 == 1:
                j = p // 2
                for h, cfg in enumerate(halves):
                    rs2p[h][j] = remote(
                        acc.at[pl.ds(cfg["off2_send"] + j * PIECE2, PIECE2)],
                        rs2_buf.at[h, pl.ds(j * PIECE2, PIECE2)],
                        m_send.at[h, RS2_0 + j], m_recv.at[h, RS2_0 + j],
                        cfg["partners"][1],
                    )
        for p in range(NPIECE // 2, NPIECE):
            q = p - NPIECE // 2
            for h, cfg in enumerate(halves):
                rs1_rdmas[h][p].wait_recv()
                row = cfg["off2_keep"] + q * CHUNK
                acc[pl.ds(row, CHUNK), :] = (
                    acc[pl.ds(row, CHUNK), :]
                    + rs1_buf[h, pl.ds(cfg["k2"] * EIGHTH + q * CHUNK,
                                       CHUNK), :]
                )

        ag1p = [[None, None], [None, None]]
        ag2a = [None, None]
        own_cp = [None, None]
        for j in range(2):
            for h, cfg in enumerate(halves):
                rs2p[h][j].wait_recv()
                row = cfg["off2_keep"] + j * PIECE2
                acc[pl.ds(row, PIECE2), :] = (
                    acc[pl.ds(row, PIECE2), :]
                    + rs2_buf[h, pl.ds(j * PIECE2, PIECE2), :]
                )
                ag1p[h][j] = remote(
                    acc.at[pl.ds(row, PIECE2)],
                    out_hbm.at[pl.ds(row, PIECE2)],
                    m_send.at[h, AG1_0 + j], m_recv.at[h, AG1_0 + j],
                    cfg["partners"][2],
                )
                if j == 1:
                    ag2a[h] = remote(
                        acc.at[pl.ds(cfg["off2_keep"], EIGHTH)],
                        out_hbm.at[pl.ds(cfg["off2_keep"], EIGHTH)],
                        m_send.at[h, AG2A], m_recv.at[h, AG2A],
                        cfg["partners"][3],
                    )
                    own_cp[h] = pltpu.make_async_copy(
                        acc.at[pl.ds(cfg["off2_keep"], EIGHTH)],
                        out_hbm.at[pl.ds(cfg["off2_keep"], EIGHTH)],
                        copy_sems.at[h],
                    )
                    own_cp[h].start()

        ag2bp = [[None, None], [None, None]]
        for j in range(2):
            for h, cfg in enumerate(halves):
                ag1p[h][j].wait_recv()
                row = cfg["off2_send"] + j * PIECE2
                ag2bp[h][j] = remote(
                    out_hbm.at[pl.ds(row, PIECE2)],
                    out_hbm.at[pl.ds(row, PIECE2)],
                    m_send.at[h, AG2B_0 + j], m_recv.at[h, AG2B_0 + j],
                    cfg["partners"][3],
                )

        for h in range(2):
            ag2a[h].wait_recv()
            for j in range(2):
                ag2bp[h][j].wait_recv()
            own_cp[h].wait()
        for h in range(2):
            for p in range(NPIECE):
                rs1_rdmas[h][p].wait_send()
            for j in range(2):
                for rdma in (rs2p[h][j], ag1p[h][j], ag2bp[h][j]):
                    rdma.wait_send()
            ag2a[h].wait_send()

    return pl.pallas_call(
        body,
        out_shape=jax.ShapeDtypeStruct((M, N), jnp.bfloat16),
        in_specs=[pl.BlockSpec(memory_space=pl.ANY)],
        out_specs=pl.BlockSpec(memory_space=pl.ANY),
        scratch_shapes=[
            pltpu.VMEM((M, N), jnp.bfloat16),
            pltpu.VMEM((2, QUART, N), jnp.bfloat16),
            pltpu.VMEM((2, EIGHTH, N), jnp.bfloat16),
            pltpu.VMEM((2, CHUNK, N), jnp.float32),
            pltpu.SemaphoreType.DMA((2,)),
            pltpu.SemaphoreType.DMA((2,)),
            pltpu.SemaphoreType.DMA((2, NPIECE)),
            pltpu.SemaphoreType.DMA((2, NPIECE)),
            pltpu.SemaphoreType.DMA((2, 7)),
            pltpu.SemaphoreType.DMA((2, 7)),
        ],
        compiler_params=pltpu.CompilerParams(
            collective_id=0,
            vmem_limit_bytes=63 * 1024 * 1024,
        ),
    )(x)
